# Initial kernel scaffold; baseline (speedup 1.0000x reference)
import jax
import jax.numpy as jnp
from jax.experimental import pallas as pl


def kernel(x, emb, W1, b1, W2, b2):
    raise NotImplementedError("write your pallas kernel here")


# speedup vs baseline: 5.2888x; 5.2888x over previous
import functools

import jax
import jax.numpy as jnp
from jax import lax
from jax.experimental import pallas as pl
from jax.experimental.pallas import tpu as pltpu
from jax.experimental.pallas import tpu_sc as plsc

EMB = 16
IDX_MINOR = 100


def _pooled_sum_sc(x, emb):
    B, L = x.shape
    V, E = emb.shape
    assert E == EMB

    info = plsc.get_sparse_core_info()
    NC, NS = info.num_cores, info.num_subcores
    NW = NC * NS
    assert B % NW == 0
    rows_per_w = B // NW
    CB = 8
    assert rows_per_w % CB == 0
    n_chunks = rows_per_w // CB
    gather_n = CB * L
    assert gather_n % IDX_MINOR == 0
    idx_rows = gather_n // IDX_MINOR

    x2d = x.reshape(B * L // IDX_MINOR, IDX_MINOR)
    rows_x2d_per_chunk = idx_rows

    mesh = plsc.VectorSubcoreMesh(core_axis_name="c", subcore_axis_name="s")

    @functools.partial(
        pl.kernel,
        out_type=jax.ShapeDtypeStruct((B, EMB), jnp.float32),
        mesh=mesh,
        scratch_types=[
            pltpu.VMEM((idx_rows, IDX_MINOR), jnp.int32),
            pltpu.VMEM((gather_n, EMB), jnp.float32),
            pltpu.VMEM((CB, EMB), jnp.float32),
            pltpu.SemaphoreType.DMA,
        ],
        compiler_params=pltpu.CompilerParams(use_tc_tiling_on_sc=False),
    )
    def sc_kernel(x_hbm, emb_hbm, out_hbm, idx_v, rows_v, pooled_v, sem):
        wid = lax.axis_index("s") * NC + lax.axis_index("c")
        row0 = wid * rows_per_w

        def chunk_body(c, _):
            base_row = pl.multiple_of(row0 + c * CB, CB)
            xrow0 = pl.multiple_of(base_row * L // IDX_MINOR, 16)
            pltpu.sync_copy(x_hbm.at[pl.ds(xrow0, rows_x2d_per_chunk)], idx_v)
            copies = []
            for j in range(idx_rows):
                copies.append(
                    pltpu.async_copy(
                        emb_hbm.at[idx_v.at[j]],
                        rows_v.at[pl.ds(j * IDX_MINOR, IDX_MINOR)],
                        sem,
                    )
                )
            for cp in copies:
                cp.wait()

            for r in range(CB):
                def add_body(i, acc):
                    return acc + rows_v[r * L + i]
                acc = lax.fori_loop(
                    0, L, add_body, jnp.zeros((EMB,), jnp.float32))
                pooled_v[r] = acc
            pltpu.sync_copy(pooled_v, out_hbm.at[pl.ds(base_row, CB)])
            return 0

        lax.fori_loop(0, n_chunks, chunk_body, 0)

    return sc_kernel(x2d, emb)


def _mlp_tc(h, W1s, b1, W2, b2):
    B = h.shape[0]
    BLK = 1024
    assert B % BLK == 0

    def body(h_ref, w1_ref, b1_ref, w2_ref, b2_ref, o_ref):
        z = jnp.dot(h_ref[...], w1_ref[...],
                    preferred_element_type=jnp.float32) + b1_ref[...]
        z = jnp.maximum(z, 0.0)
        o_ref[...] = jnp.dot(z, w2_ref[...],
                             preferred_element_type=jnp.float32) + b2_ref[...]

    return pl.pallas_call(
        body,
        grid=(B // BLK,),
        in_specs=[
            pl.BlockSpec((BLK, EMB), lambda i: (i, 0)),
            pl.BlockSpec((EMB, EMB), lambda i: (0, 0)),
            pl.BlockSpec((1, EMB), lambda i: (0, 0)),
            pl.BlockSpec((EMB, 2), lambda i: (0, 0)),
            pl.BlockSpec((1, 2), lambda i: (0, 0)),
        ],
        out_specs=pl.BlockSpec((BLK, 2), lambda i: (i, 0)),
        out_shape=jax.ShapeDtypeStruct((B, 2), jnp.float32),
    )(h, W1s, b1.reshape(1, EMB), W2, b2.reshape(1, 2))


def kernel(x, emb, W1, b1, W2, b2):
    L = x.shape[1]
    pooled = _pooled_sum_sc(x.astype(jnp.int32), emb)
    return _mlp_tc(pooled, W1 * (1.0 / L), b1, W2, b2)


# speedup vs baseline: 8.9385x; 1.6901x over previous
import functools

import jax
import jax.numpy as jnp
from jax import lax
from jax.experimental import pallas as pl
from jax.experimental.pallas import tpu as pltpu
from jax.experimental.pallas import tpu_sc as plsc

EMB = 16
IDX_MINOR = 100


def _pooled_sum_sc(x, emb):
    B, L = x.shape
    V, E = emb.shape
    assert E == EMB

    info = plsc.get_sparse_core_info()
    NC, NS = info.num_cores, info.num_subcores
    NW = NC * NS
    assert B % NW == 0
    rows_per_w = B // NW
    CB = 8
    assert rows_per_w % CB == 0
    n_chunks = rows_per_w // CB
    gather_n = CB * L
    assert gather_n % IDX_MINOR == 0
    idx_rows = gather_n // IDX_MINOR

    x2d = x.reshape(B * L // IDX_MINOR, IDX_MINOR)
    rows_x2d_per_chunk = idx_rows

    mesh = plsc.VectorSubcoreMesh(core_axis_name="c", subcore_axis_name="s")

    UNROLL = 40
    assert L % UNROLL == 0
    n_red = L // UNROLL

    @functools.partial(
        pl.kernel,
        out_type=jax.ShapeDtypeStruct((B, EMB), jnp.float32),
        mesh=mesh,
        scratch_types=[
            pltpu.VMEM((idx_rows, IDX_MINOR), jnp.int32),
            pltpu.VMEM((idx_rows, IDX_MINOR), jnp.int32),
            pltpu.VMEM((gather_n, EMB), jnp.float32),
            pltpu.VMEM((gather_n, EMB), jnp.float32),
            pltpu.VMEM((CB, EMB), jnp.float32),
            pltpu.SemaphoreType.DMA,
            pltpu.SemaphoreType.DMA,
        ],
        compiler_params=pltpu.CompilerParams(use_tc_tiling_on_sc=False),
    )
    def sc_kernel(x_hbm, emb_hbm, out_hbm,
                  idx_a, idx_b, rows_a, rows_b, pooled_v, sem_a, sem_b):
        wid = lax.axis_index("s") * NC + lax.axis_index("c")
        row0 = wid * rows_per_w

        def stage_idx(k, idx_ref):
            base_row = pl.multiple_of(row0 + k * CB, CB)
            xrow0 = pl.multiple_of(base_row * L // IDX_MINOR, 16)
            pltpu.sync_copy(x_hbm.at[pl.ds(xrow0, rows_x2d_per_chunk)], idx_ref)

        def gather_descs(idx_ref, rows_ref, sem):
            return [
                pltpu.make_async_copy(
                    emb_hbm.at[idx_ref.at[j]],
                    rows_ref.at[pl.ds(j * IDX_MINOR, IDX_MINOR)],
                    sem,
                )
                for j in range(idx_rows)
            ]

        stage_idx(0, idx_a)
        for d in gather_descs(idx_a, rows_a, sem_a):
            d.start()

        bufs = ((idx_a, rows_a, sem_a), (idx_b, rows_b, sem_b))

        def pair_body(g, _):
            for b in range(2):
                k = 2 * g + b
                idx_c, rows_c, sem_c = bufs[b]
                idx_n, rows_n, sem_n = bufs[1 - b]

                @pl.when(k + 1 < n_chunks)
                def _prefetch():
                    stage_idx(k + 1, idx_n)
                    for d in gather_descs(idx_n, rows_n, sem_n):
                        d.start()

                for d in gather_descs(idx_c, rows_c, sem_c):
                    d.wait()

                for r in range(CB):
                    base = r * L

                    def red_body(i, accs, base=base):
                        a0, a1, a2, a3 = accs
                        off = base + i * UNROLL
                        for j in range(UNROLL):
                            v = rows_c[off + j]
                            if j % 4 == 0:
                                a0 = a0 + v
                            elif j % 4 == 1:
                                a1 = a1 + v
                            elif j % 4 == 2:
                                a2 = a2 + v
                            else:
                                a3 = a3 + v
                        return (a0, a1, a2, a3)

                    z = jnp.zeros((EMB,), jnp.float32)
                    a0, a1, a2, a3 = lax.fori_loop(
                        0, n_red, red_body, (z, z, z, z))
                    pooled_v[r] = (a0 + a1) + (a2 + a3)

                base_row = pl.multiple_of(row0 + k * CB, CB)
                pltpu.sync_copy(pooled_v, out_hbm.at[pl.ds(base_row, CB)])
            return 0

        lax.fori_loop(0, n_chunks // 2, pair_body, 0)

    return sc_kernel(x2d, emb)


def _mlp_tc(h, W1s, b1, W2, b2):
    B = h.shape[0]
    BLK = 1024
    assert B % BLK == 0

    def body(h_ref, w1_ref, b1_ref, w2_ref, b2_ref, o_ref):
        z = jnp.dot(h_ref[...], w1_ref[...],
                    preferred_element_type=jnp.float32) + b1_ref[...]
        z = jnp.maximum(z, 0.0)
        o_ref[...] = jnp.dot(z, w2_ref[...],
                             preferred_element_type=jnp.float32) + b2_ref[...]

    return pl.pallas_call(
        body,
        grid=(B // BLK,),
        in_specs=[
            pl.BlockSpec((BLK, EMB), lambda i: (i, 0)),
            pl.BlockSpec((EMB, EMB), lambda i: (0, 0)),
            pl.BlockSpec((1, EMB), lambda i: (0, 0)),
            pl.BlockSpec((EMB, 2), lambda i: (0, 0)),
            pl.BlockSpec((1, 2), lambda i: (0, 0)),
        ],
        out_specs=pl.BlockSpec((BLK, 2), lambda i: (i, 0)),
        out_shape=jax.ShapeDtypeStruct((B, 2), jnp.float32),
    )(h, W1s, b1.reshape(1, EMB), W2, b2.reshape(1, 2))


def kernel(x, emb, W1, b1, W2, b2):
    L = x.shape[1]
    pooled = _pooled_sum_sc(x.astype(jnp.int32), emb)
    return _mlp_tc(pooled, W1 * (1.0 / L), b1, W2, b2)


# speedup vs baseline: 11.2732x; 1.2612x over previous
import functools

import jax
import jax.numpy as jnp
from jax import lax
from jax.experimental import pallas as pl
from jax.experimental.pallas import tpu as pltpu
from jax.experimental.pallas import tpu_sc as plsc

EMB = 16


def _detile16(at, W=8192):
    R, C = at.shape
    G = 128 // R
    grid = (C + W - 1) // W

    def body(a_ref, o_ref):
        t3 = a_ref[...].T.reshape(W // G, G, R)
        o_ref[...] = jnp.concatenate([t3[:, g, :] for g in range(G)], axis=-1)

    return pl.pallas_call(
        body, grid=(grid,),
        in_specs=[pl.BlockSpec((R, W), lambda i: (0, i))],
        out_specs=pl.BlockSpec((W * R // 128, 128), lambda i: (i, 0)),
        out_shape=jax.ShapeDtypeStruct((C * R // 128, 128), at.dtype),
    )(at)


def _pooled_sum_sc(x, emb):
    B, L = x.shape
    V, E = emb.shape
    assert E == EMB

    info = plsc.get_sparse_core_info()
    NC, NS = info.num_cores, info.num_subcores
    NW = NC * NS
    rows_per_w = B // NW
    CB = 8
    n_chunks = rows_per_w // CB
    gather_n = CB * L
    H0, H1 = 104, 96
    assert H0 + H1 == L and H0 % 8 == 0 and L % 8 == 0

    mesh = plsc.VectorSubcoreMesh(core_axis_name="c", subcore_axis_name="s")

    UNROLL = 40
    n_red = L // UNROLL

    @functools.partial(
        pl.kernel,
        out_type=jax.ShapeDtypeStruct((B, EMB), jnp.float32),
        mesh=mesh,
        scratch_types=[
            pltpu.VMEM((CB, L), jnp.int32),
            pltpu.VMEM((CB, L), jnp.int32),
            pltpu.VMEM((gather_n, EMB), jnp.float32),
            pltpu.VMEM((gather_n, EMB), jnp.float32),
            pltpu.VMEM((CB, EMB), jnp.float32),
            pltpu.SemaphoreType.DMA,
            pltpu.SemaphoreType.DMA,
        ],
        compiler_params=pltpu.CompilerParams(use_tc_tiling_on_sc=False),
    )
    def sc_kernel(x_hbm, emb_hbm, out_hbm,
                  idx_a, idx_b, rows_a, rows_b, pooled_v, sem_a, sem_b):
        wid = lax.axis_index("s") * NC + lax.axis_index("c")
        row0 = wid * rows_per_w

        def stage_idx(k, idx_ref):
            pltpu.sync_copy(x_hbm.at[pl.ds(row0 + k * CB, CB)], idx_ref)

        def gather_descs(idx_ref, rows_ref, sem):
            descs = []
            for r in range(CB):
                for off, n in ((0, H0), (H0, H1)):
                    descs.append(pltpu.make_async_copy(
                        emb_hbm.at[idx_ref.at[r, pl.ds(off, n)]],
                        rows_ref.at[pl.ds(r * L + off, n)],
                        sem,
                    ))
            return descs

        stage_idx(0, idx_a)
        for d in gather_descs(idx_a, rows_a, sem_a):
            d.start()

        bufs = ((idx_a, rows_a, sem_a), (idx_b, rows_b, sem_b))

        def pair_body(g, _):
            for b in range(2):
                k = 2 * g + b
                idx_c, rows_c, sem_c = bufs[b]
                idx_n, rows_n, sem_n = bufs[1 - b]

                @pl.when(k + 1 < n_chunks)
                def _prefetch():
                    stage_idx(k + 1, idx_n)
                    for d in gather_descs(idx_n, rows_n, sem_n):
                        d.start()

                for d in gather_descs(idx_c, rows_c, sem_c):
                    d.wait()

                for r in range(CB):
                    base = r * L

                    def red_body(i, accs, base=base):
                        a0, a1, a2, a3 = accs
                        off = base + i * UNROLL
                        for j in range(UNROLL):
                            v = rows_c[off + j]
                            if j % 4 == 0:
                                a0 = a0 + v
                            elif j % 4 == 1:
                                a1 = a1 + v
                            elif j % 4 == 2:
                                a2 = a2 + v
                            else:
                                a3 = a3 + v
                        return (a0, a1, a2, a3)

                    z = jnp.zeros((EMB,), jnp.float32)
                    a0, a1, a2, a3 = lax.fori_loop(
                        0, n_red, red_body, (z, z, z, z))
                    pooled_v[r] = (a0 + a1) + (a2 + a3)

                pltpu.sync_copy(pooled_v,
                                out_hbm.at[pl.ds(row0 + k * CB, CB)])
            return 0

        lax.fori_loop(0, n_chunks // 2, pair_body, 0)

    return sc_kernel(x, emb)


def _mlp_tc(h, W1s, b1, W2, b2):
    B = h.shape[0]
    BLK = 1024

    def body(h_ref, w1_ref, b1_ref, w2_ref, b2_ref, o_ref):
        z = jnp.dot(h_ref[...], w1_ref[...],
                    preferred_element_type=jnp.float32) + b1_ref[...]
        z = jnp.maximum(z, 0.0)
        o_ref[...] = jnp.dot(z, w2_ref[...],
                             preferred_element_type=jnp.float32) + b2_ref[...]

    return pl.pallas_call(
        body,
        grid=(B // BLK,),
        in_specs=[
            pl.BlockSpec((BLK, EMB), lambda i: (i, 0)),
            pl.BlockSpec((EMB, EMB), lambda i: (0, 0)),
            pl.BlockSpec((1, EMB), lambda i: (0, 0)),
            pl.BlockSpec((EMB, 2), lambda i: (0, 0)),
            pl.BlockSpec((1, 2), lambda i: (0, 0)),
        ],
        out_specs=pl.BlockSpec((BLK, 2), lambda i: (i, 0)),
        out_shape=jax.ShapeDtypeStruct((B, 2), jnp.float32),
    )(h, W1s, b1.reshape(1, EMB), W2, b2.reshape(1, 2))


def kernel(x, emb, W1, b1, W2, b2):
    B, L = x.shape
    V = emb.shape[0]
    emb_lin = _detile16(emb.T)
    pooled = _pooled_sum_sc(x.astype(jnp.int32),
                            emb_lin.reshape(V, EMB))
    return _mlp_tc(pooled, W1 * (1.0 / L), b1, W2, b2)


# speedup vs baseline: 14.4839x; 1.2848x over previous
import functools

import jax
import jax.numpy as jnp
from jax import lax
from jax.experimental import pallas as pl
from jax.experimental.pallas import tpu as pltpu
from jax.experimental.pallas import tpu_sc as plsc

EMB = 16


_DW = 8192
_DS = _DW // 8


def _detile16(at):
    R, C = at.shape
    G = 128 // R
    grid = (C + _DW - 1) // _DW

    def body(a_ref, o_ref):
        ri = lax.broadcasted_iota(jnp.int32, (R, 128), 0)
        ci = lax.broadcasted_iota(jnp.int32, (R, 128), 1)

        def compute(a):
            acc = None
            for g in range(G):
                E = (ci == ri + R * g).astype(jnp.float32)
                d = jax.lax.dot_general(
                    a[:, g * _DS:(g + 1) * _DS], E, (((0,), (0,)), ((), ())),
                    preferred_element_type=jnp.float32)
                acc = d if acc is None else acc + d
            return acc

        pid = pl.program_id(0)

        @pl.when(pid != grid - 1)
        def _full():
            o_ref[...] = compute(a_ref[...])

        @pl.when(pid == grid - 1)
        def _tail():
            cols = lax.broadcasted_iota(jnp.int32, (R, _DW), 1) + pid * _DW
            a = jnp.where(cols < C, a_ref[...], 0.0)
            o_ref[...] = compute(a)

    return pl.pallas_call(
        body, grid=(grid,),
        in_specs=[pl.BlockSpec((R, _DW), lambda i: (0, i))],
        out_specs=pl.BlockSpec((_DS, 128), lambda i: (i, 0)),
        out_shape=jax.ShapeDtypeStruct((grid * _DS, 128), at.dtype),
    )(at)


def _remap_x(xt):
    R, C = xt.shape
    W = 2048

    def body(a_ref, o_ref):
        k = a_ref[...]
        e = jnp.bitwise_and(k, _DW - 1)
        o_ref[...] = (k - e) | ((e & (_DS - 1)) << 3) | (e >> 10)

    return pl.pallas_call(
        body, grid=(C // W,),
        in_specs=[pl.BlockSpec((R, W), lambda i: (0, i))],
        out_specs=pl.BlockSpec((R, W), lambda i: (0, i)),
        out_shape=jax.ShapeDtypeStruct((R, C), xt.dtype),
    )(xt)


def _pooled_sum_sc(x, emb):
    B, L = x.shape
    V, E = emb.shape
    assert E == EMB

    info = plsc.get_sparse_core_info()
    NC, NS = info.num_cores, info.num_subcores
    NW = NC * NS
    rows_per_w = B // NW
    CB = 8
    n_chunks = rows_per_w // CB
    gather_n = CB * L
    H0, H1 = 104, 96
    assert H0 + H1 == L and H0 % 8 == 0 and L % 8 == 0

    mesh = plsc.VectorSubcoreMesh(core_axis_name="c", subcore_axis_name="s")

    UNROLL = 40
    n_red = L // UNROLL

    @functools.partial(
        pl.kernel,
        out_type=jax.ShapeDtypeStruct((B, EMB), jnp.float32),
        mesh=mesh,
        scratch_types=[
            pltpu.VMEM((CB, L), jnp.int32),
            pltpu.VMEM((CB, L), jnp.int32),
            pltpu.VMEM((gather_n, EMB), jnp.float32),
            pltpu.VMEM((gather_n, EMB), jnp.float32),
            pltpu.VMEM((CB, EMB), jnp.float32),
            pltpu.SemaphoreType.DMA,
            pltpu.SemaphoreType.DMA,
        ],
        compiler_params=pltpu.CompilerParams(use_tc_tiling_on_sc=False),
    )
    def sc_kernel(x_hbm, emb_hbm, out_hbm,
                  idx_a, idx_b, rows_a, rows_b, pooled_v, sem_a, sem_b):
        wid = lax.axis_index("s") * NC + lax.axis_index("c")
        row0 = wid * rows_per_w

        def stage_idx(k, idx_ref):
            pltpu.sync_copy(x_hbm.at[pl.ds(row0 + k * CB, CB)], idx_ref)

        def gather_descs(idx_ref, rows_ref, sem):
            descs = []
            for r in range(CB):
                for off, n in ((0, H0), (H0, H1)):
                    descs.append(pltpu.make_async_copy(
                        emb_hbm.at[idx_ref.at[r, pl.ds(off, n)]],
                        rows_ref.at[pl.ds(r * L + off, n)],
                        sem,
                    ))
            return descs

        stage_idx(0, idx_a)
        for d in gather_descs(idx_a, rows_a, sem_a):
            d.start()

        bufs = ((idx_a, rows_a, sem_a), (idx_b, rows_b, sem_b))

        def pair_body(g, _):
            for b in range(2):
                k = 2 * g + b
                idx_c, rows_c, sem_c = bufs[b]
                idx_n, rows_n, sem_n = bufs[1 - b]

                @pl.when(k + 1 < n_chunks)
                def _prefetch():
                    stage_idx(k + 1, idx_n)
                    for d in gather_descs(idx_n, rows_n, sem_n):
                        d.start()

                for d in gather_descs(idx_c, rows_c, sem_c):
                    d.wait()

                for r in range(CB):
                    base = r * L

                    def red_body(i, accs, base=base):
                        a0, a1, a2, a3 = accs
                        off = base + i * UNROLL
                        for j in range(UNROLL):
                            v = rows_c[off + j]
                            if j % 4 == 0:
                                a0 = a0 + v
                            elif j % 4 == 1:
                                a1 = a1 + v
                            elif j % 4 == 2:
                                a2 = a2 + v
                            else:
                                a3 = a3 + v
                        return (a0, a1, a2, a3)

                    z = jnp.zeros((EMB,), jnp.float32)
                    a0, a1, a2, a3 = lax.fori_loop(
                        0, n_red, red_body, (z, z, z, z))
                    pooled_v[r] = (a0 + a1) + (a2 + a3)

                pltpu.sync_copy(pooled_v,
                                out_hbm.at[pl.ds(row0 + k * CB, CB)])
            return 0

        lax.fori_loop(0, n_chunks // 2, pair_body, 0)

    return sc_kernel(x, emb)


def _mlp_tc(h, W1s, b1, W2, b2):
    B = h.shape[0]
    BLK = 1024

    def body(h_ref, w1_ref, b1_ref, w2_ref, b2_ref, o_ref):
        z = jnp.dot(h_ref[...], w1_ref[...],
                    preferred_element_type=jnp.float32) + b1_ref[...]
        z = jnp.maximum(z, 0.0)
        o_ref[...] = jnp.dot(z, w2_ref[...],
                             preferred_element_type=jnp.float32) + b2_ref[...]

    return pl.pallas_call(
        body,
        grid=(B // BLK,),
        in_specs=[
            pl.BlockSpec((BLK, EMB), lambda i: (i, 0)),
            pl.BlockSpec((EMB, EMB), lambda i: (0, 0)),
            pl.BlockSpec((1, EMB), lambda i: (0, 0)),
            pl.BlockSpec((EMB, 2), lambda i: (0, 0)),
            pl.BlockSpec((1, 2), lambda i: (0, 0)),
        ],
        out_specs=pl.BlockSpec((BLK, 2), lambda i: (i, 0)),
        out_shape=jax.ShapeDtypeStruct((B, 2), jnp.float32),
    )(h, W1s, b1.reshape(1, EMB), W2, b2.reshape(1, 2))


def kernel(x, emb, W1, b1, W2, b2):
    B, L = x.shape
    emb_lin = _detile16(emb.T)
    Vp = emb_lin.shape[0] * 128 // EMB
    x_remap = _remap_x(x.T.astype(jnp.int32)).T
    pooled = _pooled_sum_sc(x_remap, emb_lin.reshape(Vp, EMB))
    return _mlp_tc(pooled, W1 * (1.0 / L), b1, W2, b2)


# speedup vs baseline: 19.4900x; 1.3456x over previous
import functools

import jax
import jax.numpy as jnp
from jax import lax
from jax.experimental import pallas as pl
from jax.experimental.pallas import tpu as pltpu
from jax.experimental.pallas import tpu_sc as plsc

EMB = 16


_DW = 16384
_DS = _DW // 8
_DSH = 11


def _detile16(at):
    R, C = at.shape
    G = 128 // R
    grid = (C + _DW - 1) // _DW

    def body(a_ref, o_ref):
        ri = lax.broadcasted_iota(jnp.int32, (128, 128), 0)
        ci = lax.broadcasted_iota(jnp.int32, (128, 128), 1)
        eye = (ri == ci).astype(jnp.float32)

        def compute(a):
            ap = jnp.concatenate(
                [a[:, g * _DS:(g + 1) * _DS] for g in range(G)], axis=0)
            return jax.lax.dot_general(
                ap, eye, (((0,), (0,)), ((), ())),
                preferred_element_type=jnp.float32)

        pid = pl.program_id(0)

        @pl.when(pid != grid - 1)
        def _full():
            o_ref[...] = compute(a_ref[...])

        @pl.when(pid == grid - 1)
        def _tail():
            cols = lax.broadcasted_iota(jnp.int32, (R, _DW), 1) + pid * _DW
            a = jnp.where(cols < C, a_ref[...], 0.0)
            o_ref[...] = compute(a)

    return pl.pallas_call(
        body, grid=(grid,),
        in_specs=[pl.BlockSpec((R, _DW), lambda i: (0, i))],
        out_specs=pl.BlockSpec((_DS, 128), lambda i: (i, 0)),
        out_shape=jax.ShapeDtypeStruct((grid * _DS, 128), at.dtype),
    )(at)


def _remap_x(xt):
    R, C = xt.shape
    W = 2048

    def body(a_ref, o_ref):
        k = a_ref[...]
        e = jnp.bitwise_and(k, _DW - 1)
        o_ref[...] = (k - e) | ((e & (_DS - 1)) << 3) | (e >> _DSH)

    return pl.pallas_call(
        body, grid=(C // W,),
        in_specs=[pl.BlockSpec((R, W), lambda i: (0, i))],
        out_specs=pl.BlockSpec((R, W), lambda i: (0, i)),
        out_shape=jax.ShapeDtypeStruct((R, C), xt.dtype),
    )(xt)


def _pooled_sum_sc(x, emb):
    B, L = x.shape
    V, E = emb.shape
    assert E == EMB

    info = plsc.get_sparse_core_info()
    NC, NS = info.num_cores, info.num_subcores
    NW = NC * NS
    rows_per_w = B // NW
    CB = 8
    n_chunks = rows_per_w // CB
    gather_n = CB * L
    H0, H1 = 104, 96
    assert H0 + H1 == L and H0 % 8 == 0 and L % 8 == 0

    mesh = plsc.VectorSubcoreMesh(core_axis_name="c", subcore_axis_name="s")

    UNROLL = 40
    n_red = L // UNROLL

    @functools.partial(
        pl.kernel,
        out_type=jax.ShapeDtypeStruct((B, EMB), jnp.float32),
        mesh=mesh,
        scratch_types=[
            pltpu.VMEM((CB, L), jnp.int32),
            pltpu.VMEM((CB, L), jnp.int32),
            pltpu.VMEM((gather_n, EMB), jnp.float32),
            pltpu.VMEM((gather_n, EMB), jnp.float32),
            pltpu.VMEM((CB, EMB), jnp.float32),
            pltpu.SemaphoreType.DMA,
            pltpu.SemaphoreType.DMA,
        ],
        compiler_params=pltpu.CompilerParams(use_tc_tiling_on_sc=False),
    )
    def sc_kernel(x_hbm, emb_hbm, out_hbm,
                  idx_a, idx_b, rows_a, rows_b, pooled_v, sem_a, sem_b):
        wid = lax.axis_index("s") * NC + lax.axis_index("c")
        row0 = wid * rows_per_w

        def stage_idx(k, idx_ref):
            pltpu.sync_copy(x_hbm.at[pl.ds(row0 + k * CB, CB)], idx_ref)

        def gather_descs(idx_ref, rows_ref, sem):
            descs = []
            for r in range(CB):
                for off, n in ((0, H0), (H0, H1)):
                    descs.append(pltpu.make_async_copy(
                        emb_hbm.at[idx_ref.at[r, pl.ds(off, n)]],
                        rows_ref.at[pl.ds(r * L + off, n)],
                        sem,
                    ))
            return descs

        stage_idx(0, idx_a)
        for d in gather_descs(idx_a, rows_a, sem_a):
            d.start()

        bufs = ((idx_a, rows_a, sem_a), (idx_b, rows_b, sem_b))

        def pair_body(g, _):
            for b in range(2):
                k = 2 * g + b
                idx_c, rows_c, sem_c = bufs[b]
                idx_n, rows_n, sem_n = bufs[1 - b]

                @pl.when(k + 1 < n_chunks)
                def _prefetch():
                    stage_idx(k + 1, idx_n)
                    for d in gather_descs(idx_n, rows_n, sem_n):
                        d.start()

                for d in gather_descs(idx_c, rows_c, sem_c):
                    d.wait()

                for r in range(CB):
                    base = r * L

                    def red_body(i, accs, base=base):
                        a0, a1, a2, a3 = accs
                        off = base + i * UNROLL
                        for j in range(UNROLL):
                            v = rows_c[off + j]
                            if j % 4 == 0:
                                a0 = a0 + v
                            elif j % 4 == 1:
                                a1 = a1 + v
                            elif j % 4 == 2:
                                a2 = a2 + v
                            else:
                                a3 = a3 + v
                        return (a0, a1, a2, a3)

                    z = jnp.zeros((EMB,), jnp.float32)
                    a0, a1, a2, a3 = lax.fori_loop(
                        0, n_red, red_body, (z, z, z, z))
                    pooled_v[r] = (a0 + a1) + (a2 + a3)

                pltpu.sync_copy(pooled_v,
                                out_hbm.at[pl.ds(row0 + k * CB, CB)])
            return 0

        lax.fori_loop(0, n_chunks // 2, pair_body, 0)

    return sc_kernel(x, emb)


def _mlp_tc(h2d, W1s, b1, W2, b2):
    Bd8 = h2d.shape[0]
    BLK = 1024
    eye8 = jnp.eye(8, dtype=jnp.float32)
    W1d = jnp.kron(eye8, W1s)
    b1d = jnp.tile(b1, 8).reshape(1, 128)
    W2d = jnp.kron(eye8, W2)
    b2d = jnp.tile(b2, 8).reshape(1, 16)

    def body(h_ref, w1_ref, b1_ref, w2_ref, b2_ref, o_ref):
        z = jnp.dot(h_ref[...], w1_ref[...],
                    preferred_element_type=jnp.float32) + b1_ref[...]
        z = jnp.maximum(z, 0.0)
        o_ref[...] = jnp.dot(z, w2_ref[...],
                             preferred_element_type=jnp.float32) + b2_ref[...]

    out = pl.pallas_call(
        body,
        grid=(Bd8 // BLK,),
        in_specs=[
            pl.BlockSpec((BLK, 128), lambda i: (i, 0)),
            pl.BlockSpec((128, 128), lambda i: (0, 0)),
            pl.BlockSpec((1, 128), lambda i: (0, 0)),
            pl.BlockSpec((128, EMB), lambda i: (0, 0)),
            pl.BlockSpec((1, EMB), lambda i: (0, 0)),
        ],
        out_specs=pl.BlockSpec((BLK, EMB), lambda i: (i, 0)),
        out_shape=jax.ShapeDtypeStruct((Bd8, EMB), jnp.float32),
    )(h2d, W1d, b1d, W2d, b2d)
    return out.reshape(Bd8 * 8, 2)


def kernel(x, emb, W1, b1, W2, b2):
    B, L = x.shape
    emb_lin = _detile16(emb.T)
    Vp = emb_lin.shape[0] * 128 // EMB
    x_remap = _remap_x(x.T.astype(jnp.int32)).T
    pooled = _pooled_sum_sc(x_remap, emb_lin.reshape(Vp, EMB))
    return _mlp_tc(pooled.reshape(B // 8, 128),
                   W1 * (1.0 / L), b1, W2, b2)
